# speedup vs baseline: 3.0671x; 3.0671x over previous
import functools

import jax
import jax.numpy as jnp
from jax.experimental import pallas as pl
from jax.experimental.pallas import tpu as pltpu

_N = 100000
_H = 256
_B = 2000
_NB = _N // _B


def _table_body(emb_ref, w1_ref, out_ref):
    out_ref[...] = jax.lax.dot_general(
        emb_ref[...], w1_ref[...], (((1,), (1,)), ((), ())),
        preferred_element_type=jnp.float32)


def _main_body(x_ref, chi_ref, b_ref, table_ref, w2_ref, w3_ref, blin_ref,
               w11_ref, v_ref, v1_ref):
    x = x_ref[0, 0, :]
    onehot = (jax.lax.broadcasted_iota(jnp.int32, (_B, 128), 1)
              == x[:, None]).astype(jnp.float32)
    xe = jax.lax.dot_general(onehot, table_ref[...], (((1,), (0,)), ((), ())),
                             preferred_element_type=jnp.float32)
    t2 = jax.lax.dot_general(chi_ref[...], w2_ref[...], (((1,), (1,)), ((), ())),
                             preferred_element_type=jnp.float32)
    t3 = jax.lax.dot_general(b_ref[...], w3_ref[...], (((1,), (1,)), ((), ())),
                             preferred_element_type=jnp.float32)
    v_pre = xe + t2 + t3 + blin_ref[...]
    v = v_pre * jax.nn.sigmoid(v_pre)
    v_ref[...] = v
    v1_ref[...] = jax.lax.dot_general(v, w11_ref[...], (((1,), (1,)), ((), ())),
                                      preferred_element_type=jnp.float32)


@jax.jit
def kernel(x, chi, b, emb, W_lin, b_lin, W_lin1):
    x = x.astype(jnp.int32)
    emb_pad = jnp.pad(emb, ((0, 128 - emb.shape[0]), (0, 0)))
    W1 = W_lin[:, :_H]
    W2 = W_lin[:, _H:2 * _H]
    W3 = W_lin[:, 2 * _H:]
    table = pl.pallas_call(
        _table_body,
        out_shape=jax.ShapeDtypeStruct((128, _H), jnp.float32),
    )(emb_pad, W1)
    x3 = x.reshape(_NB, 1, _B)
    blin2 = b_lin.reshape(1, _H)
    v, v1 = pl.pallas_call(
        _main_body,
        grid=(_NB,),
        in_specs=[
            pl.BlockSpec((1, 1, _B), lambda i: (i, 0, 0)),
            pl.BlockSpec((_B, _H), lambda i: (i, 0)),
            pl.BlockSpec((_B, _H), lambda i: (i, 0)),
            pl.BlockSpec((128, _H), lambda i: (0, 0)),
            pl.BlockSpec((_H, _H), lambda i: (0, 0)),
            pl.BlockSpec((_H, _H), lambda i: (0, 0)),
            pl.BlockSpec((1, _H), lambda i: (0, 0)),
            pl.BlockSpec((_H, _H), lambda i: (0, 0)),
        ],
        out_specs=[
            pl.BlockSpec((_B, _H), lambda i: (i, 0)),
            pl.BlockSpec((_B, _H), lambda i: (i, 0)),
        ],
        out_shape=[
            jax.ShapeDtypeStruct((_N, _H), jnp.float32),
            jax.ShapeDtypeStruct((_N, _H), jnp.float32),
        ],
        compiler_params=pltpu.CompilerParams(
            dimension_semantics=("arbitrary",)),
    )(x3, chi, b, table, W2, W3, blin2, W_lin1)
    return (v, v1)
